# baseline (speedup 1.0000x reference)
import functools

import jax
import jax.numpy as jnp
from jax import lax
from jax.experimental import pallas as pl
from jax.experimental.pallas import tpu as pltpu
from jax.experimental.pallas import tpu_sc as plsc

B = 4096
L = 200
F = 4
NC, NS = 2, 16
NW = NC * NS
RPW = B // NW
R = 32
NCHUNK = RPW // R
LPAD = 208
NGRP = (L + 15) // 16
WIDTHS = (12, 31, 7, 24)

_mesh = plsc.VectorSubcoreMesh(core_axis_name="c", subcore_axis_name="s")


@functools.partial(
    pl.kernel,
    out_type=jax.ShapeDtypeStruct((B, L), jnp.float32),
    mesh=_mesh,
    compiler_params=pltpu.CompilerParams(
        needs_layout_passes=False, use_tc_tiling_on_sc=False),
    scratch_types=[
        pltpu.VMEM((R, L, F), jnp.int32),
        pltpu.VMEM((R, L, F), jnp.int32),
        pltpu.VMEM((R, LPAD), jnp.float32),
        pltpu.VMEM((R, LPAD), jnp.float32),
        pltpu.SemaphoreType.DMA,
        pltpu.SemaphoreType.DMA,
        pltpu.SemaphoreType.DMA,
        pltpu.SemaphoreType.DMA,
    ],
)
def _t2v_sc(x_hbm, out_hbm, in0, in1, ob0, ob1, si0, si1, so0, so1):
    wid = lax.axis_index("s") * NC + lax.axis_index("c")
    row_base = wid * RPW
    inbufs, obufs = (in0, in1), (ob0, ob1)
    isems, osems = (si0, si1), (so0, so1)

    iota = lax.iota(jnp.int32, 16)
    lane_lt8 = iota < 8
    one = jnp.full((16,), 1.0, jnp.float32)
    zero = jnp.full((16,), 0.0, jnp.float32)

    def start_in(c):
        return pltpu.async_copy(
            x_hbm.at[pl.ds(row_base + c * R, R)],
            inbufs[c % 2], isems[c % 2])

    def start_out(c):
        return pltpu.async_copy(
            obufs[c % 2].at[:, pl.ds(0, L)],
            out_hbm.at[pl.ds(row_base + c * R, R)],
            osems[c % 2])

    def process(c):
        ib, ob = inbufs[c % 2], obufs[c % 2]

        def row_body(r, carry):
            acc = zero
            for j in range(NGRP):
                if j == NGRP - 1:
                    idx_l = jnp.minimum(iota + j * 16, L - 1)
                else:
                    idx_l = iota + j * 16
                idx_r = jnp.full((16,), r, jnp.int32)
                cnt = zero
                for f, w in enumerate(WIDTHS):
                    idx_f = jnp.full((16,), f, jnp.int32)
                    v = plsc.load_gather(ib, [idx_r, idx_l, idx_f])
                    vu = plsc.bitcast(v, jnp.uint32)
                    cnt = cnt + jnp.where(vu < jnp.uint32(w), one, zero)
                if j == NGRP - 1:
                    cnt = jnp.where(lane_lt8, cnt, zero)
                acc = acc + cnt * cnt
                ob[r, pl.ds(j * 16, 16)] = cnt
            t = jnp.full((16,), jnp.sum(acc), jnp.float32)
            gi = jnp.int32(0x5F3759DF) - (plsc.bitcast(t, jnp.int32) >> 1)
            g = plsc.bitcast(gi, jnp.float32)
            for _ in range(3):
                g = g * (1.5 - 0.5 * t * g * g)
            for j in range(NGRP):
                ob[r, pl.ds(j * 16, 16)] = ob[r, pl.ds(j * 16, 16)] * g
            return carry

        lax.fori_loop(0, R, row_body, 0)

    cp_in = [None] * NCHUNK
    cp_out = [None] * NCHUNK
    cp_in[0] = start_in(0)
    for c in range(NCHUNK):
        if c + 1 < NCHUNK:
            cp_in[c + 1] = start_in(c + 1)
        cp_in[c].wait()
        if c >= 2:
            cp_out[c - 2].wait()
        process(c)
        cp_out[c] = start_out(c)
    cp_out[NCHUNK - 2].wait()
    cp_out[NCHUNK - 1].wait()


def kernel(x):
    return _t2v_sc(x.astype(jnp.int32))


# speedup vs baseline: 18.8618x; 18.8618x over previous
import functools

import jax
import jax.numpy as jnp
from jax import lax
from jax.experimental import pallas as pl
from jax.experimental.pallas import tpu as pltpu
from jax.experimental.pallas import tpu_sc as plsc

B = 4096
L = 200
F = 4
NC, NS = 2, 16
NW = NC * NS
CW = B // NW
LCH = 40
NCHUNK = L // LCH
NV = CW // 16
WIDTHS = (12, 31, 7, 24)

_mesh = plsc.VectorSubcoreMesh(core_axis_name="c", subcore_axis_name="s")


@functools.partial(
    pl.kernel,
    out_type=jax.ShapeDtypeStruct((L, B), jnp.float32),
    mesh=_mesh,
    compiler_params=pltpu.CompilerParams(needs_layout_passes=False),
    scratch_types=[
        pltpu.VMEM((LCH, CW), jnp.int32),
        pltpu.VMEM((LCH, CW), jnp.int32),
        pltpu.VMEM((LCH, CW), jnp.int32),
        pltpu.VMEM((LCH, CW), jnp.int32),
        pltpu.VMEM((LCH, CW), jnp.int32),
        pltpu.VMEM((LCH, CW), jnp.int32),
        pltpu.VMEM((LCH, CW), jnp.int32),
        pltpu.VMEM((LCH, CW), jnp.int32),
        pltpu.VMEM((L, CW), jnp.float32),
        pltpu.SemaphoreType.DMA,
        pltpu.SemaphoreType.DMA,
        pltpu.SemaphoreType.DMA,
    ],
)
def _t2v_sc(m_hbm, d_hbm, w_hbm, h_hbm, out_hbm,
            b00, b01, b02, b03, b10, b11, b12, b13, cnt_v, si0, si1, so):
    wid = lax.axis_index("s") * NC + lax.axis_index("c")
    col = wid * CW
    field_hbm = (m_hbm, d_hbm, w_hbm, h_hbm)
    inbufs = ((b00, b01, b02, b03), (b10, b11, b12, b13))
    isems = (si0, si1)

    one = jnp.full((16,), 1.0, jnp.float32)
    zero = jnp.full((16,), 0.0, jnp.float32)

    def start_in(c):
        ib = inbufs[c % 2]
        return [
            pltpu.async_copy(
                field_hbm[f].at[pl.ds(c * LCH, LCH), pl.ds(col, CW)],
                ib[f], isems[c % 2])
            for f in range(F)
        ]

    def process(c, acc):
        ib = inbufs[c % 2]

        def row_body(r, acc):
            out_acc = []
            for v in range(NV):
                sl = pl.ds(v * 16, 16)
                cnt = zero
                for f, w in enumerate(WIDTHS):
                    vu = plsc.bitcast(ib[f][r, sl], jnp.uint32)
                    cnt = cnt + jnp.where(vu < jnp.uint32(w), one, zero)
                cnt_v[c * LCH + r, sl] = cnt
                out_acc.append(acc[v] + cnt * cnt)
            return tuple(out_acc)

        return lax.fori_loop(0, LCH, row_body, acc)

    cps = [None] * NCHUNK
    cps[0] = start_in(0)
    acc = (zero,) * NV
    for c in range(NCHUNK):
        if c + 1 < NCHUNK:
            cps[c + 1] = start_in(c + 1)
        for cp in cps[c]:
            cp.wait()
        acc = process(c, acc)

    gs = []
    for v in range(NV):
        t = acc[v]
        gi = jnp.int32(0x5F3759DF) - (plsc.bitcast(t, jnp.int32) >> 1)
        g = plsc.bitcast(gi, jnp.float32)
        for _ in range(3):
            g = g * (1.5 - 0.5 * t * g * g)
        gs.append(g)

    def scale_body(r, carry):
        for v in range(NV):
            sl = pl.ds(v * 16, 16)
            cnt_v[r, sl] = cnt_v[r, sl] * gs[v]
        return carry

    lax.fori_loop(0, L, scale_body, 0)
    pltpu.async_copy(cnt_v, out_hbm.at[:, pl.ds(col, CW)], so).wait()


def kernel(x):
    x = x.astype(jnp.int32)
    planes = [x[:, :, f].T for f in range(F)]
    return _t2v_sc(*planes).T


# speedup vs baseline: 26.8510x; 1.4236x over previous
import functools

import jax
import jax.numpy as jnp
from jax import lax
from jax.experimental import pallas as pl
from jax.experimental.pallas import tpu as pltpu
from jax.experimental.pallas import tpu_sc as plsc

B = 4096
L = 200
F = 4
NC, NS = 2, 16
NW = NC * NS
CW = B // NW
LCH = 40
NCHUNK = L // LCH
NV = CW // 16
WIDTHS = (12, 31, 7, 24)

_mesh = plsc.VectorSubcoreMesh(core_axis_name="c", subcore_axis_name="s")


@functools.partial(
    pl.kernel,
    out_type=jax.ShapeDtypeStruct((L, B), jnp.float32),
    mesh=_mesh,
    compiler_params=pltpu.CompilerParams(needs_layout_passes=False),
    scratch_types=[
        pltpu.VMEM((LCH * F, CW), jnp.int32),
        pltpu.VMEM((LCH * F, CW), jnp.int32),
        pltpu.VMEM((L, CW), jnp.float32),
        pltpu.SemaphoreType.DMA,
        pltpu.SemaphoreType.DMA,
        pltpu.SemaphoreType.DMA,
    ],
)
def _t2v_sc(xs_hbm, out_hbm, in0, in1, cnt_v, si0, si1, so):
    wid = lax.axis_index("s") * NC + lax.axis_index("c")
    col = wid * CW
    inbufs, isems = (in0, in1), (si0, si1)

    one = jnp.full((16,), 1.0, jnp.float32)
    zero = jnp.full((16,), 0.0, jnp.float32)

    def start_in(c):
        return pltpu.async_copy(
            xs_hbm.at[pl.ds(c * LCH * F, LCH * F), pl.ds(col, CW)],
            inbufs[c % 2], isems[c % 2])

    def process(c, acc):
        ib = inbufs[c % 2]

        def row_body(r, acc):
            r4 = r * F
            out_acc = []
            for v in range(NV):
                sl = pl.ds(v * 16, 16)
                cnt = zero
                for f, w in enumerate(WIDTHS):
                    vu = plsc.bitcast(ib[r4 + f, sl], jnp.uint32)
                    cnt = cnt + jnp.where(vu < jnp.uint32(w), one, zero)
                cnt_v[c * LCH + r, sl] = cnt
                out_acc.append(acc[v] + cnt * cnt)
            return tuple(out_acc)

        return lax.fori_loop(0, LCH, row_body, acc)

    cps = [None] * NCHUNK
    cps[0] = start_in(0)
    acc = (zero,) * NV
    for c in range(NCHUNK):
        if c + 1 < NCHUNK:
            cps[c + 1] = start_in(c + 1)
        cps[c].wait()
        acc = process(c, acc)

    gs = []
    for v in range(NV):
        t = acc[v]
        gi = jnp.int32(0x5F3759DF) - (plsc.bitcast(t, jnp.int32) >> 1)
        g = plsc.bitcast(gi, jnp.float32)
        for _ in range(3):
            g = g * (1.5 - 0.5 * t * g * g)
        gs.append(g)

    def scale_body(r, carry):
        for v in range(NV):
            sl = pl.ds(v * 16, 16)
            cnt_v[r, sl] = cnt_v[r, sl] * gs[v]
        return carry

    lax.fori_loop(0, L, scale_body, 0)
    pltpu.async_copy(cnt_v, out_hbm.at[:, pl.ds(col, CW)], so).wait()


def kernel(x):
    xs = jnp.transpose(x.astype(jnp.int32), (1, 2, 0)).reshape(L * F, B)
    return _t2v_sc(xs).T


# speedup vs baseline: 35.3609x; 1.3169x over previous
import functools

import jax
import jax.numpy as jnp
from jax import lax
from jax.experimental import pallas as pl
from jax.experimental.pallas import tpu as pltpu
from jax.experimental.pallas import tpu_sc as plsc

B = 4096
L = 200
F = 4
NC, NS = 2, 16
NW = NC * NS
CW = B // NW
LCH = 40
NCHUNK = L // LCH
NV = CW // 16
WIDTHS = (12, 31, 7, 24)

_mesh = plsc.VectorSubcoreMesh(core_axis_name="c", subcore_axis_name="s")


@functools.partial(
    pl.kernel,
    out_type=jax.ShapeDtypeStruct((L, B), jnp.float32),
    mesh=_mesh,
    compiler_params=pltpu.CompilerParams(needs_layout_passes=False),
    scratch_types=[
        pltpu.VMEM((LCH, 1, F, CW), jnp.int32),
        pltpu.VMEM((LCH, 1, F, CW), jnp.int32),
        pltpu.VMEM((L, CW), jnp.float32),
        pltpu.SemaphoreType.DMA,
        pltpu.SemaphoreType.DMA,
        pltpu.SemaphoreType.DMA,
    ],
)
def _t2v_sc(xs_hbm, out_hbm, in0, in1, cnt_v, si0, si1, so):
    wid = lax.axis_index("s") * NC + lax.axis_index("c")
    col = wid * CW
    inbufs, isems = (in0, in1), (si0, si1)

    one = jnp.full((16,), 1.0, jnp.float32)
    zero = jnp.full((16,), 0.0, jnp.float32)

    def start_in(c):
        return pltpu.async_copy(
            xs_hbm.at[pl.ds(c * LCH, LCH), pl.ds(wid, 1)],
            inbufs[c % 2], isems[c % 2])

    def process(c, acc):
        ib = inbufs[c % 2]

        def row_body(r, acc):
            out_acc = []
            for v in range(NV):
                sl = pl.ds(v * 16, 16)
                cnt = zero
                for f, w in enumerate(WIDTHS):
                    vu = plsc.bitcast(ib[r, 0, f, sl], jnp.uint32)
                    cnt = cnt + jnp.where(vu < jnp.uint32(w), one, zero)
                cnt_v[c * LCH + r, sl] = cnt
                out_acc.append(acc[v] + cnt * cnt)
            return tuple(out_acc)

        return lax.fori_loop(0, LCH, row_body, acc)

    cps = [None] * NCHUNK
    cps[0] = start_in(0)
    acc = (zero,) * NV
    for c in range(NCHUNK):
        if c + 1 < NCHUNK:
            cps[c + 1] = start_in(c + 1)
        cps[c].wait()
        acc = process(c, acc)

    gs = []
    for v in range(NV):
        t = acc[v]
        gi = jnp.int32(0x5F3759DF) - (plsc.bitcast(t, jnp.int32) >> 1)
        g = plsc.bitcast(gi, jnp.float32)
        for _ in range(3):
            g = g * (1.5 - 0.5 * t * g * g)
        gs.append(g)

    def scale_body(r, carry):
        for v in range(NV):
            sl = pl.ds(v * 16, 16)
            cnt_v[r, sl] = cnt_v[r, sl] * gs[v]
        return carry

    lax.fori_loop(0, L, scale_body, 0)
    pltpu.async_copy(cnt_v, out_hbm.at[:, pl.ds(col, CW)], so).wait()


def kernel(x):
    xs = x.astype(jnp.int32).reshape(NW, CW, L, F).transpose(2, 0, 3, 1)
    return _t2v_sc(xs).T
